# baseline (speedup 1.0000x reference)
import jax
import jax.numpy as jnp
from jax import lax
from jax.experimental import pallas as pl
from jax.experimental.pallas import tpu as pltpu
from jax.experimental.pallas import tpu_sc as plsc
import functools

_N = 10000
_E = 160000
_CHUNK = 128
_B = 128
_NTILES = 16
_RPT = 624
_COMB = 104
_TAIL_BASE = _NTILES * _RPT
_TAIL = _N - _TAIL_BASE
_NBATCH = _E // _B
_MP_NORM = 0.1


def _zero_rowbuf(rowbuf):
    def body(t, _):
        z = jnp.zeros((16,), jnp.float32)
        for k in range(8):
            rowbuf[t, pl.ds(k * 16, 16)] = z
        return 0
    lax.fori_loop(0, _B, body, 0)


def _process_chunk(chunk, nf_ref, out_ref, acc, rowbuf, accbuf,
                   ibuf, fbuf, ivbuf, pfbuf, mcbuf,
                   gsem, idx_hbm, flt_hbm, sid):
    r0 = 2 * chunk

    _zero_rowbuf(rowbuf)
    row_base = sid * _RPT
    for j in range(6):
        pltpu.sync_copy(rowbuf.at[pl.ds(0, _COMB)],
                        acc.at[pl.ds(row_base + j * _COMB, _COMB)])

    @pl.when(sid == _NTILES - 1)
    def _():
        pltpu.sync_copy(rowbuf.at[pl.ds(0, _TAIL)],
                        acc.at[pl.ds(_TAIL_BASE, _TAIL)])

    plsc.subcore_barrier()

    ivs = [[ivbuf[g, r0 + rr] for g in range(2)] for rr in range(2)]
    pfs = [[pfbuf[g, r0 + rr] for g in range(2)] for rr in range(2)]

    def edge_body(e, _):
        elv = jnp.full((16,), fbuf[0, pl.ds(e, 16)][0], jnp.float32)
        cfv = jnp.full((16,), fbuf[1, pl.ds(e, 16)][0], jnp.float32)
        for rr in range(2):
            w0 = jnp.exp(elv * ivs[rr][0]) * (cfv * pfs[rr][0])
            w1 = jnp.exp(elv * ivs[rr][1]) * (cfv * pfs[rr][1])
            base = rr * 64
            rowbuf[e, pl.ds(base, 16)] = rowbuf[e, pl.ds(base, 16)] * w0
            for a in range(1, 4):
                col = base + a * 16
                rowbuf[e, pl.ds(col, 16)] = rowbuf[e, pl.ds(col, 16)] * w1
        return 0

    def batch_body(i, _):
        bi = sid + i * _NTILES
        pltpu.sync_copy(idx_hbm.at[bi], ibuf)
        pltpu.sync_copy(flt_hbm.at[bi], fbuf.at[pl.ds(0, 2)])
        pltpu.async_copy(nf_ref.at[ibuf.at[0]], rowbuf, gsem).wait()
        lax.fori_loop(0, _B, edge_body, 0)
        pltpu.sync_copy(rowbuf, acc.at[ibuf.at[1]], add=True)
        return 0

    nb = 78 + jnp.where(sid < 2, 1, 0)
    lax.fori_loop(0, nb, batch_body, 0)
    plsc.subcore_barrier()

    mcvals = []
    for rr in range(2):
        row = []
        for a in range(4):
            g = 0 if a == 0 else 1
            row.append(mcbuf[g, r0 + rr])
        mcvals.append(row)

    def comb_body(t, _):
        for rr in range(2):
            for a in range(4):
                col = rr * 64 + a * 16
                nfv = rowbuf[t, pl.ds(col, 16)]
                av = accbuf[t, pl.ds(col, 16)]
                rowbuf[t, pl.ds(col, 16)] = nfv * mcvals[rr][a] + av * _MP_NORM
        return 0

    def combine(row0, nrows):
        pltpu.sync_copy(nf_ref.at[pl.ds(row0, nrows)], rowbuf.at[pl.ds(0, nrows)])
        pltpu.sync_copy(acc.at[pl.ds(row0, nrows)], accbuf.at[pl.ds(0, nrows)])
        lax.fori_loop(0, nrows, comb_body, 0)
        pltpu.sync_copy(rowbuf.at[pl.ds(0, nrows)], out_ref.at[pl.ds(row0, nrows)])

    for j in range(6):
        combine(row_base + j * _COMB, _COMB)

    @pl.when(sid == _NTILES - 1)
    def _():
        combine(_TAIL_BASE, _TAIL)

    plsc.subcore_barrier()


def _sc_body(idx_hbm, flt_hbm, iv_hbm, pf_hbm, mc_hbm,
             nf0, nf1, nf2, nf3, out_hbm, acc, rowbuf, accbuf,
             ibuf, fbuf, ivbuf, pfbuf, mcbuf, gsem):
    cid = lax.axis_index("c")
    sid = lax.axis_index("s")

    pltpu.sync_copy(iv_hbm, ivbuf)
    pltpu.sync_copy(pf_hbm, pfbuf)
    pltpu.sync_copy(mc_hbm, mcbuf)

    common = dict(acc=acc, rowbuf=rowbuf, accbuf=accbuf, ibuf=ibuf,
                  fbuf=fbuf, ivbuf=ivbuf, pfbuf=pfbuf, mcbuf=mcbuf,
                  gsem=gsem, idx_hbm=idx_hbm, flt_hbm=flt_hbm, sid=sid)

    @pl.when(cid == 0)
    def _():
        _process_chunk(0, nf0, out_hbm.at[0], **common)
        _process_chunk(1, nf1, out_hbm.at[1], **common)

    @pl.when(cid == 1)
    def _():
        _process_chunk(2, nf2, out_hbm.at[2], **common)
        _process_chunk(3, nf3, out_hbm.at[3], **common)


@jax.jit
def kernel(node_feat, edge_lengths, radial_cutoff_fn, edge_index,
           prefactor, invr0, memory_coef):
    n = node_feat.shape[0]
    nfc = node_feat.reshape(n, 4, _CHUNK)
    chunks = [nfc[:, k, :] for k in range(4)]
    idxp = jnp.transpose(edge_index.astype(jnp.int32).reshape(2, _NBATCH, _B),
                         (1, 0, 2))
    fltp = jnp.transpose(
        jnp.stack([edge_lengths, radial_cutoff_fn]).reshape(2, _NBATCH, _B),
        (1, 0, 2))

    mesh = plsc.VectorSubcoreMesh(core_axis_name="c", subcore_axis_name="s")
    run = pl.kernel(
        _sc_body,
        out_type=jax.ShapeDtypeStruct((4, n, _CHUNK), jnp.float32),
        mesh=mesh,
        scratch_types=[
            pltpu.VMEM_SHARED((_N, _CHUNK), jnp.float32),
            pltpu.VMEM((_B, _CHUNK), jnp.float32),
            pltpu.VMEM((_COMB, _CHUNK), jnp.float32),
            pltpu.VMEM((2, _B), jnp.int32),
            pltpu.VMEM((3, _B), jnp.float32),
            pltpu.VMEM((2, 8, 16), jnp.float32),
            pltpu.VMEM((2, 8, 16), jnp.float32),
            pltpu.VMEM((2, 8, 16), jnp.float32),
            pltpu.SemaphoreType.DMA,
        ],
    )
    out = run(idxp, fltp, -invr0, prefactor, memory_coef,
              chunks[0], chunks[1], chunks[2], chunks[3])
    return jnp.transpose(out, (1, 0, 2)).reshape(n, 8, 4, 16)
